# Initial kernel scaffold; baseline (speedup 1.0000x reference)
import functools

import jax
import jax.numpy as jnp
from jax import lax
from jax.experimental import pallas as pl
from jax.experimental.pallas import tpu as pltpu
from jax.experimental.pallas import tpu_sc as plsc

N = 10000
N_PAD = 10240
IN_DIM = 128
HID = 64
OUT = 16
E = 320000
NC = 2
NS = 16
NW = NC * NS
CHUNK = 128
E_PER_W = E // NW
CHUNKS = -(-E_PER_W // CHUNK)
E_PAD_W = CHUNKS * CHUNK
ROWS_PER_TILE = N_PAD // NS

_mesh = plsc.VectorSubcoreMesh(core_axis_name="c", subcore_axis_name="s")


@functools.partial(
    pl.kernel,
    out_type=jax.ShapeDtypeStruct((NC, N_PAD), jnp.float32),
    mesh=_mesh,
    scratch_types=[
        pltpu.VMEM((CHUNKS, CHUNK), jnp.int32),
        pltpu.VMEM((CHUNK,), jnp.float32),
        pltpu.VMEM_SHARED((N_PAD,), jnp.float32),
    ],
)
def _sc_degree(dst_hbm, zeros_hbm, out_hbm, dst_v, ones_v, acc):
    c = lax.axis_index("c")
    s = lax.axis_index("s")
    wid = c * NS + s
    r0 = s * ROWS_PER_TILE
    pltpu.sync_copy(zeros_hbm.at[pl.ds(r0, ROWS_PER_TILE)],
                    acc.at[pl.ds(r0, ROWS_PER_TILE)])
    pltpu.sync_copy(dst_hbm.at[wid], dst_v)
    for i in range(CHUNK // 16):
        ones_v[pl.ds(i * 16, 16)] = jnp.full((16,), 1.0, jnp.float32)
    plsc.subcore_barrier()

    def body(j, carry):
        pltpu.sync_copy(ones_v, acc.at[dst_v.at[j]], add=True)
        return carry

    lax.fori_loop(0, CHUNKS, body, 0)
    plsc.subcore_barrier()
    pltpu.sync_copy(acc.at[pl.ds(r0, ROWS_PER_TILE)],
                    out_hbm.at[c, pl.ds(r0, ROWS_PER_TILE)])


def _make_agg(D):
    @functools.partial(
        pl.kernel,
        out_type=jax.ShapeDtypeStruct((NC, N_PAD, D), jnp.float32),
        mesh=_mesh,
        scratch_types=[
            pltpu.VMEM((CHUNKS, CHUNK), jnp.int32),
            pltpu.VMEM((CHUNKS, CHUNK), jnp.int32),
            pltpu.VMEM((CHUNK, D), jnp.float32),
            pltpu.VMEM_SHARED((N_PAD, D), jnp.float32),
            pltpu.SemaphoreType.DMA,
        ],
    )
    def agg(h_hbm, src_hbm, dst_hbm, zeros_hbm, out_hbm,
            src_v, dst_v, rows_v, acc, sem):
        c = lax.axis_index("c")
        s = lax.axis_index("s")
        wid = c * NS + s
        r0 = s * ROWS_PER_TILE
        pltpu.sync_copy(zeros_hbm.at[pl.ds(r0, ROWS_PER_TILE)],
                        acc.at[pl.ds(r0, ROWS_PER_TILE)])
        pltpu.sync_copy(src_hbm.at[wid], src_v)
        pltpu.sync_copy(dst_hbm.at[wid], dst_v)
        plsc.subcore_barrier()

        def body(j, carry):
            pltpu.async_copy(h_hbm.at[src_v.at[j]], rows_v, sem).wait()
            pltpu.sync_copy(rows_v, acc.at[dst_v.at[j]], add=True)
            return carry

        lax.fori_loop(0, CHUNKS, body, 0)
        plsc.subcore_barrier()
        pltpu.sync_copy(acc.at[pl.ds(r0, ROWS_PER_TILE)],
                        out_hbm.at[c, pl.ds(r0, ROWS_PER_TILE)])

    return agg


_agg_hid = _make_agg(HID)
_agg_out = _make_agg(OUT)


_BR = 1024


def _dinv_col(deg_ref):
    deg = deg_ref[:, 0:1] + deg_ref[:, 1:2] + 1.0
    return lax.rsqrt(deg)


def _tc1_body(deg_ref, x_ref, w_ref, o_ref):
    dinv = _dinv_col(deg_ref)
    h = jnp.dot(x_ref[...], w_ref[...], preferred_element_type=jnp.float32)
    o_ref[...] = h * dinv


def _tc1(degt, xp, W1):
    return pl.pallas_call(
        _tc1_body,
        grid=(N_PAD // _BR,),
        in_specs=[
            pl.BlockSpec((_BR, 2), lambda i: (i, 0)),
            pl.BlockSpec((_BR, IN_DIM), lambda i: (i, 0)),
            pl.BlockSpec((IN_DIM, HID), lambda i: (0, 0)),
        ],
        out_specs=pl.BlockSpec((_BR, HID), lambda i: (i, 0)),
        out_shape=jax.ShapeDtypeStruct((N_PAD, HID), jnp.float32),
    )(degt, xp, W1)


def _tc2_body(p_ref, h_ref, deg_ref, b_ref, w_ref, o_ref):
    dinv = _dinv_col(deg_ref)
    agg = p_ref[0] + p_ref[1] + h_ref[...]
    z = jnp.maximum(agg * dinv + b_ref[...], 0.0)
    h2 = jnp.dot(z, w_ref[...], preferred_element_type=jnp.float32)
    o_ref[...] = h2 * dinv


def _tc2(p1, h1s, degt, b1, W2):
    return pl.pallas_call(
        _tc2_body,
        grid=(N_PAD // _BR,),
        in_specs=[
            pl.BlockSpec((NC, _BR, HID), lambda i: (0, i, 0)),
            pl.BlockSpec((_BR, HID), lambda i: (i, 0)),
            pl.BlockSpec((_BR, 2), lambda i: (i, 0)),
            pl.BlockSpec((1, HID), lambda i: (0, 0)),
            pl.BlockSpec((HID, OUT), lambda i: (0, 0)),
        ],
        out_specs=pl.BlockSpec((_BR, OUT), lambda i: (i, 0)),
        out_shape=jax.ShapeDtypeStruct((N_PAD, OUT), jnp.float32),
    )(p1, h1s, degt, b1, W2)


def _tc3_body(p_ref, h_ref, deg_ref, b_ref, o_ref):
    dinv = _dinv_col(deg_ref)
    o = (p_ref[0] + p_ref[1] + h_ref[...]) * dinv + b_ref[...]
    m = jnp.max(o, axis=1, keepdims=True)
    lse = jnp.log(jnp.sum(jnp.exp(o - m), axis=1, keepdims=True)) + m
    o_ref[...] = o - lse


def _tc3(p2, h2s, degt, b2):
    return pl.pallas_call(
        _tc3_body,
        grid=(N_PAD // _BR,),
        in_specs=[
            pl.BlockSpec((NC, _BR, OUT), lambda i: (0, i, 0)),
            pl.BlockSpec((_BR, OUT), lambda i: (i, 0)),
            pl.BlockSpec((_BR, 2), lambda i: (i, 0)),
            pl.BlockSpec((1, OUT), lambda i: (0, 0)),
        ],
        out_specs=pl.BlockSpec((_BR, OUT), lambda i: (i, 0)),
        out_shape=jax.ShapeDtypeStruct((N_PAD, OUT), jnp.float32),
    )(p2, h2s, degt, b2)


def _edge_layout(a):
    a = a.reshape(NW, E_PER_W)
    a = jnp.pad(a, ((0, 0), (0, E_PAD_W - E_PER_W)),
                constant_values=N_PAD - 1)
    return a.reshape(NW, CHUNKS, CHUNK)


@jax.jit
def kernel(x, edge_index, W1, b1, W2, b2):
    ei = edge_index.astype(jnp.int32)
    src3 = _edge_layout(ei[0])
    dst3 = _edge_layout(ei[1])
    xp = jnp.pad(x, ((0, N_PAD - N), (0, 0)))
    z1 = jnp.zeros((N_PAD,), jnp.float32)
    zh = jnp.zeros((N_PAD, HID), jnp.float32)
    zo = jnp.zeros((N_PAD, OUT), jnp.float32)

    degp = _sc_degree(dst3, z1)
    degt = degp.T
    h1s = _tc1(degt, xp, W1)
    p1 = _agg_hid(h1s, src3, dst3, zh)
    h2s = _tc2(p1, h1s, degt, b1.reshape(1, HID), W2)
    p2 = _agg_out(h2s, src3, dst3, zo)
    o = _tc3(p2, h2s, degt, b2.reshape(1, OUT))
    return o[:N]


# speedup vs baseline: 24.9696x; 24.9696x over previous
import functools

import jax
import jax.numpy as jnp
from jax import lax
from jax.experimental import pallas as pl
from jax.experimental.pallas import tpu as pltpu
from jax.experimental.pallas import tpu_sc as plsc

N = 10000
N_PAD = 10240
IN_DIM = 128
HID = 64
OUT = 16
E = 320000
NC = 2
NS = 16
NW = NC * NS
CHUNK = 128
E_PER_W = E // NW
CHUNKS = -(-E_PER_W // CHUNK)
E_PAD_W = CHUNKS * CHUNK
ROWS_PER_TILE = N_PAD // NS

_mesh = plsc.VectorSubcoreMesh(core_axis_name="c", subcore_axis_name="s")


@functools.partial(
    pl.kernel,
    out_type=jax.ShapeDtypeStruct((NC, N_PAD), jnp.float32),
    mesh=_mesh,
    scratch_types=[
        pltpu.VMEM((CHUNKS, CHUNK), jnp.int32),
        pltpu.VMEM((CHUNK,), jnp.float32),
        pltpu.VMEM_SHARED((N_PAD,), jnp.float32),
    ],
)
def _sc_degree(dst_hbm, zeros_hbm, out_hbm, dst_v, ones_v, acc):
    c = lax.axis_index("c")
    s = lax.axis_index("s")
    wid = c * NS + s
    r0 = s * ROWS_PER_TILE
    pltpu.sync_copy(zeros_hbm.at[pl.ds(r0, ROWS_PER_TILE)],
                    acc.at[pl.ds(r0, ROWS_PER_TILE)])
    pltpu.sync_copy(dst_hbm.at[wid], dst_v)
    for i in range(CHUNK // 16):
        ones_v[pl.ds(i * 16, 16)] = jnp.full((16,), 1.0, jnp.float32)
    plsc.subcore_barrier()

    def body(j, carry):
        pltpu.sync_copy(ones_v, acc.at[dst_v.at[j]], add=True)
        return carry

    lax.fori_loop(0, CHUNKS, body, 0)
    plsc.subcore_barrier()
    pltpu.sync_copy(acc.at[pl.ds(r0, ROWS_PER_TILE)],
                    out_hbm.at[c, pl.ds(r0, ROWS_PER_TILE)])


def _make_agg(D):
    @functools.partial(
        pl.kernel,
        out_type=jax.ShapeDtypeStruct((NC, N_PAD, D), jnp.float32),
        mesh=_mesh,
        scratch_types=[
            pltpu.VMEM((CHUNKS, CHUNK), jnp.int32),
            pltpu.VMEM((CHUNKS, CHUNK), jnp.int32),
            pltpu.VMEM((CHUNK, D), jnp.float32),
            pltpu.VMEM_SHARED((N_PAD, D), jnp.float32),
            pltpu.SemaphoreType.DMA,
        ],
        compiler_params=pltpu.CompilerParams(use_tc_tiling_on_sc=False),
    )
    def agg(h_hbm, src_hbm, dst_hbm, zeros_hbm, out_hbm,
            src_v, dst_v, rows_v, acc, sem):
        c = lax.axis_index("c")
        s = lax.axis_index("s")
        wid = c * NS + s
        r0 = s * ROWS_PER_TILE
        pltpu.sync_copy(zeros_hbm.at[pl.ds(r0, ROWS_PER_TILE)],
                        acc.at[pl.ds(r0, ROWS_PER_TILE)])
        pltpu.sync_copy(src_hbm.at[wid], src_v)
        pltpu.sync_copy(dst_hbm.at[wid], dst_v)
        plsc.subcore_barrier()

        def body(j, carry):
            pltpu.async_copy(h_hbm.at[src_v.at[j]], rows_v, sem).wait()
            pltpu.sync_copy(rows_v, acc.at[dst_v.at[j]], add=True)
            return carry

        lax.fori_loop(0, CHUNKS, body, 0)
        plsc.subcore_barrier()
        pltpu.sync_copy(acc.at[pl.ds(r0, ROWS_PER_TILE)],
                        out_hbm.at[c, pl.ds(r0, ROWS_PER_TILE)])

    return agg


_agg_hid = _make_agg(HID)
_agg_out = _make_agg(OUT)


_BR = 1024


def _dinv_col(deg_ref):
    deg = deg_ref[:, 0:1] + deg_ref[:, 1:2] + 1.0
    return lax.rsqrt(deg)


def _tc1_body(deg_ref, x_ref, w_ref, o_ref):
    dinv = _dinv_col(deg_ref)
    h = jnp.dot(x_ref[...], w_ref[...], preferred_element_type=jnp.float32)
    o_ref[...] = h * dinv


def _tc1(degt, xp, W1):
    return pl.pallas_call(
        _tc1_body,
        grid=(N_PAD // _BR,),
        in_specs=[
            pl.BlockSpec((_BR, 2), lambda i: (i, 0)),
            pl.BlockSpec((_BR, IN_DIM), lambda i: (i, 0)),
            pl.BlockSpec((IN_DIM, HID), lambda i: (0, 0)),
        ],
        out_specs=pl.BlockSpec((_BR, HID), lambda i: (i, 0)),
        out_shape=jax.ShapeDtypeStruct((N_PAD, HID), jnp.float32),
    )(degt, xp, W1)


def _tc2_body(p_ref, h_ref, deg_ref, b_ref, w_ref, o_ref):
    dinv = _dinv_col(deg_ref)
    agg = p_ref[0] + p_ref[1] + h_ref[...]
    z = jnp.maximum(agg * dinv + b_ref[...], 0.0)
    h2 = jnp.dot(z, w_ref[...], preferred_element_type=jnp.float32)
    o_ref[...] = h2 * dinv


def _tc2(p1, h1s, degt, b1, W2):
    return pl.pallas_call(
        _tc2_body,
        grid=(N_PAD // _BR,),
        in_specs=[
            pl.BlockSpec((NC, _BR, HID), lambda i: (0, i, 0)),
            pl.BlockSpec((_BR, HID), lambda i: (i, 0)),
            pl.BlockSpec((_BR, 2), lambda i: (i, 0)),
            pl.BlockSpec((1, HID), lambda i: (0, 0)),
            pl.BlockSpec((HID, OUT), lambda i: (0, 0)),
        ],
        out_specs=pl.BlockSpec((_BR, OUT), lambda i: (i, 0)),
        out_shape=jax.ShapeDtypeStruct((N_PAD, OUT), jnp.float32),
    )(p1, h1s, degt, b1, W2)


def _tc3_body(p_ref, h_ref, deg_ref, b_ref, o_ref):
    dinv = _dinv_col(deg_ref)
    o = (p_ref[0] + p_ref[1] + h_ref[...]) * dinv + b_ref[...]
    m = jnp.max(o, axis=1, keepdims=True)
    lse = jnp.log(jnp.sum(jnp.exp(o - m), axis=1, keepdims=True)) + m
    o_ref[...] = o - lse


def _tc3(p2, h2s, degt, b2):
    return pl.pallas_call(
        _tc3_body,
        grid=(N_PAD // _BR,),
        in_specs=[
            pl.BlockSpec((NC, _BR, OUT), lambda i: (0, i, 0)),
            pl.BlockSpec((_BR, OUT), lambda i: (i, 0)),
            pl.BlockSpec((_BR, 2), lambda i: (i, 0)),
            pl.BlockSpec((1, OUT), lambda i: (0, 0)),
        ],
        out_specs=pl.BlockSpec((_BR, OUT), lambda i: (i, 0)),
        out_shape=jax.ShapeDtypeStruct((N_PAD, OUT), jnp.float32),
    )(p2, h2s, degt, b2)


def _edge_layout(a):
    a = a.reshape(NW, E_PER_W)
    a = jnp.pad(a, ((0, 0), (0, E_PAD_W - E_PER_W)),
                constant_values=N_PAD - 1)
    return a.reshape(NW, CHUNKS, CHUNK)


@jax.jit
def kernel(x, edge_index, W1, b1, W2, b2):
    ei = edge_index.astype(jnp.int32)
    src3 = _edge_layout(ei[0])
    dst3 = _edge_layout(ei[1])
    xp = jnp.pad(x, ((0, N_PAD - N), (0, 0)))
    z1 = jnp.zeros((N_PAD,), jnp.float32)
    zh = jnp.zeros((N_PAD, HID), jnp.float32)
    zo = jnp.zeros((N_PAD, OUT), jnp.float32)

    degp = _sc_degree(dst3, z1)
    degt = degp.T
    h1s = _tc1(degt, xp, W1)
    p1 = _agg_hid(h1s, src3, dst3, zh)
    h2s = _tc2(p1, h1s, degt, b1.reshape(1, HID), W2)
    p2 = _agg_out(h2s, src3, dst3, zo)
    o = _tc3(p2, h2s, degt, b2.reshape(1, OUT))
    return o[:N]


# speedup vs baseline: 25.2214x; 1.0101x over previous
import functools

import jax
import jax.numpy as jnp
from jax import lax
from jax.experimental import pallas as pl
from jax.experimental.pallas import tpu as pltpu
from jax.experimental.pallas import tpu_sc as plsc

N = 10000
N_PAD = 10240
IN_DIM = 128
HID = 64
OUT = 16
E = 320000
NC = 2
NS = 16
NW = NC * NS
CHUNK = 128
E_PER_W = E // NW
K = 8
CHUNKS = 80
E_PAD_W = CHUNKS * CHUNK
ROWS_PER_TILE = N_PAD // NS

_mesh = plsc.VectorSubcoreMesh(core_axis_name="c", subcore_axis_name="s")


@functools.partial(
    pl.kernel,
    out_type=jax.ShapeDtypeStruct((NC, N_PAD), jnp.float32),
    mesh=_mesh,
    scratch_types=[
        pltpu.VMEM((CHUNKS, CHUNK), jnp.int32),
        pltpu.VMEM((CHUNK,), jnp.float32),
        pltpu.VMEM_SHARED((N_PAD,), jnp.float32),
        pltpu.SemaphoreType.DMA,
    ],
)
def _sc_degree(dst_hbm, zeros_hbm, out_hbm, dst_v, ones_v, acc, sem):
    c = lax.axis_index("c")
    s = lax.axis_index("s")
    wid = c * NS + s
    r0 = s * ROWS_PER_TILE
    pltpu.sync_copy(zeros_hbm.at[pl.ds(r0, ROWS_PER_TILE)],
                    acc.at[pl.ds(r0, ROWS_PER_TILE)])
    pltpu.sync_copy(dst_hbm.at[wid], dst_v)
    for i in range(CHUNK // 16):
        ones_v[pl.ds(i * 16, 16)] = jnp.full((16,), 1.0, jnp.float32)
    plsc.subcore_barrier()

    def body(r, carry):
        ds = [pltpu.async_copy(ones_v, acc.at[dst_v.at[r * K + k]], sem,
                               add=True)
              for k in range(K)]
        for d in ds:
            d.wait()
        return carry

    lax.fori_loop(0, CHUNKS // K, body, 0)
    plsc.subcore_barrier()
    pltpu.sync_copy(acc.at[pl.ds(r0, ROWS_PER_TILE)],
                    out_hbm.at[c, pl.ds(r0, ROWS_PER_TILE)])


def _make_agg(D):
    @functools.partial(
        pl.kernel,
        out_type=jax.ShapeDtypeStruct((NC, N_PAD, D), jnp.float32),
        mesh=_mesh,
        scratch_types=[
            pltpu.VMEM((CHUNKS, CHUNK), jnp.int32),
            pltpu.VMEM((CHUNKS, CHUNK), jnp.int32),
            pltpu.VMEM((K, CHUNK, D), jnp.float32),
            pltpu.VMEM_SHARED((N_PAD, D), jnp.float32),
            pltpu.SemaphoreType.DMA,
            pltpu.SemaphoreType.DMA,
        ],
        compiler_params=pltpu.CompilerParams(use_tc_tiling_on_sc=False),
    )
    def agg(h_hbm, src_hbm, dst_hbm, zeros_hbm, out_hbm,
            src_v, dst_v, rows_v, acc, gsem, ssem):
        c = lax.axis_index("c")
        s = lax.axis_index("s")
        wid = c * NS + s
        r0 = s * ROWS_PER_TILE
        pltpu.sync_copy(zeros_hbm.at[pl.ds(r0, ROWS_PER_TILE)],
                        acc.at[pl.ds(r0, ROWS_PER_TILE)])
        pltpu.sync_copy(src_hbm.at[wid], src_v)
        pltpu.sync_copy(dst_hbm.at[wid], dst_v)
        plsc.subcore_barrier()

        def body(r, carry):
            gds = [pltpu.async_copy(h_hbm.at[src_v.at[r * K + k]],
                                    rows_v.at[k], gsem)
                   for k in range(K)]
            sds = []
            for k in range(K):
                gds[k].wait()
                sds.append(pltpu.async_copy(rows_v.at[k],
                                            acc.at[dst_v.at[r * K + k]],
                                            ssem, add=True))
            for d in sds:
                d.wait()
            return carry

        lax.fori_loop(0, CHUNKS // K, body, 0)
        plsc.subcore_barrier()
        pltpu.sync_copy(acc.at[pl.ds(r0, ROWS_PER_TILE)],
                        out_hbm.at[c, pl.ds(r0, ROWS_PER_TILE)])

    return agg


_agg_hid = _make_agg(HID)
_agg_out = _make_agg(OUT)


_BR = 1024


def _dinv_col(deg_ref):
    deg = deg_ref[:, 0:1] + deg_ref[:, 1:2] + 1.0
    return lax.rsqrt(deg)


def _tc1_body(deg_ref, x_ref, w_ref, o_ref):
    dinv = _dinv_col(deg_ref)
    h = jnp.dot(x_ref[...], w_ref[...], preferred_element_type=jnp.float32)
    o_ref[...] = h * dinv


def _tc1(degt, xp, W1):
    return pl.pallas_call(
        _tc1_body,
        grid=(N_PAD // _BR,),
        in_specs=[
            pl.BlockSpec((_BR, 2), lambda i: (i, 0)),
            pl.BlockSpec((_BR, IN_DIM), lambda i: (i, 0)),
            pl.BlockSpec((IN_DIM, HID), lambda i: (0, 0)),
        ],
        out_specs=pl.BlockSpec((_BR, HID), lambda i: (i, 0)),
        out_shape=jax.ShapeDtypeStruct((N_PAD, HID), jnp.float32),
    )(degt, xp, W1)


def _tc2_body(p_ref, h_ref, deg_ref, b_ref, w_ref, o_ref):
    dinv = _dinv_col(deg_ref)
    agg = p_ref[0] + p_ref[1] + h_ref[...]
    z = jnp.maximum(agg * dinv + b_ref[...], 0.0)
    h2 = jnp.dot(z, w_ref[...], preferred_element_type=jnp.float32)
    o_ref[...] = h2 * dinv


def _tc2(p1, h1s, degt, b1, W2):
    return pl.pallas_call(
        _tc2_body,
        grid=(N_PAD // _BR,),
        in_specs=[
            pl.BlockSpec((NC, _BR, HID), lambda i: (0, i, 0)),
            pl.BlockSpec((_BR, HID), lambda i: (i, 0)),
            pl.BlockSpec((_BR, 2), lambda i: (i, 0)),
            pl.BlockSpec((1, HID), lambda i: (0, 0)),
            pl.BlockSpec((HID, OUT), lambda i: (0, 0)),
        ],
        out_specs=pl.BlockSpec((_BR, OUT), lambda i: (i, 0)),
        out_shape=jax.ShapeDtypeStruct((N_PAD, OUT), jnp.float32),
    )(p1, h1s, degt, b1, W2)


def _tc3_body(p_ref, h_ref, deg_ref, b_ref, o_ref):
    dinv = _dinv_col(deg_ref)
    o = (p_ref[0] + p_ref[1] + h_ref[...]) * dinv + b_ref[...]
    m = jnp.max(o, axis=1, keepdims=True)
    lse = jnp.log(jnp.sum(jnp.exp(o - m), axis=1, keepdims=True)) + m
    o_ref[...] = o - lse


def _tc3(p2, h2s, degt, b2):
    return pl.pallas_call(
        _tc3_body,
        grid=(N_PAD // _BR,),
        in_specs=[
            pl.BlockSpec((NC, _BR, OUT), lambda i: (0, i, 0)),
            pl.BlockSpec((_BR, OUT), lambda i: (i, 0)),
            pl.BlockSpec((_BR, 2), lambda i: (i, 0)),
            pl.BlockSpec((1, OUT), lambda i: (0, 0)),
        ],
        out_specs=pl.BlockSpec((_BR, OUT), lambda i: (i, 0)),
        out_shape=jax.ShapeDtypeStruct((N_PAD, OUT), jnp.float32),
    )(p2, h2s, degt, b2)


def _edge_layout(a):
    a = a.reshape(NW, E_PER_W)
    a = jnp.pad(a, ((0, 0), (0, E_PAD_W - E_PER_W)),
                constant_values=N_PAD - 1)
    return a.reshape(NW, CHUNKS, CHUNK)


@jax.jit
def kernel(x, edge_index, W1, b1, W2, b2):
    ei = edge_index.astype(jnp.int32)
    src3 = _edge_layout(ei[0])
    dst3 = _edge_layout(ei[1])
    xp = jnp.pad(x, ((0, N_PAD - N), (0, 0)))
    z1 = jnp.zeros((N_PAD,), jnp.float32)
    zh = jnp.zeros((N_PAD, HID), jnp.float32)
    zo = jnp.zeros((N_PAD, OUT), jnp.float32)

    degp = _sc_degree(dst3, z1)
    degt = degp.T
    h1s = _tc1(degt, xp, W1)
    p1 = _agg_hid(h1s, src3, dst3, zh)
    h2s = _tc2(p1, h1s, degt, b1.reshape(1, HID), W2)
    p2 = _agg_out(h2s, src3, dst3, zo)
    o = _tc3(p2, h2s, degt, b2.reshape(1, OUT))
    return o[:N]


# speedup vs baseline: 36.5998x; 1.4511x over previous
import functools

import jax
import jax.numpy as jnp
from jax import lax
from jax.experimental import pallas as pl
from jax.experimental.pallas import tpu as pltpu
from jax.experimental.pallas import tpu_sc as plsc

N = 10000
N_PAD = 10240
IN_DIM = 128
HID = 64
OUT = 16
E = 320000
NC = 2
NS = 16
NW = NC * NS
CHUNK = 128
E_PER_W = E // NW
K = 8
CHUNKS = 80
E_PAD_W = CHUNKS * CHUNK
ROWS_PER_TILE = N_PAD // NS

_mesh = plsc.VectorSubcoreMesh(core_axis_name="c", subcore_axis_name="s")


@functools.partial(
    pl.kernel,
    out_type=jax.ShapeDtypeStruct((NC, N_PAD), jnp.float32),
    mesh=_mesh,
    scratch_types=[
        pltpu.VMEM((CHUNKS, CHUNK), jnp.int32),
        pltpu.VMEM((CHUNK,), jnp.float32),
        pltpu.VMEM_SHARED((N_PAD,), jnp.float32),
        pltpu.SemaphoreType.DMA,
    ],
)
def _sc_degree(dst_hbm, zeros_hbm, out_hbm, dst_v, ones_v, acc, sem):
    c = lax.axis_index("c")
    s = lax.axis_index("s")
    wid = c * NS + s
    r0 = s * ROWS_PER_TILE
    pltpu.sync_copy(zeros_hbm.at[pl.ds(r0, ROWS_PER_TILE)],
                    acc.at[pl.ds(r0, ROWS_PER_TILE)])
    pltpu.sync_copy(dst_hbm.at[wid], dst_v)
    for i in range(CHUNK // 16):
        ones_v[pl.ds(i * 16, 16)] = jnp.full((16,), 1.0, jnp.float32)
    plsc.subcore_barrier()

    def body(r, carry):
        ds = [pltpu.async_copy(ones_v, acc.at[dst_v.at[r * K + k]], sem,
                               add=True)
              for k in range(K)]
        for d in ds:
            d.wait()
        return carry

    lax.fori_loop(0, CHUNKS // K, body, 0)
    plsc.subcore_barrier()
    pltpu.sync_copy(acc.at[pl.ds(r0, ROWS_PER_TILE)],
                    out_hbm.at[c, pl.ds(r0, ROWS_PER_TILE)])


def _make_agg(D, table_in_spmem, K=K):
    @functools.partial(
        pl.kernel,
        out_type=jax.ShapeDtypeStruct((NC, N_PAD, D), jnp.float32),
        mesh=_mesh,
        scratch_types=[
            pltpu.VMEM((CHUNKS, CHUNK), jnp.int32),
            pltpu.VMEM((CHUNKS, CHUNK), jnp.int32),
            pltpu.VMEM((K, CHUNK, D), jnp.float32),
            pltpu.VMEM_SHARED((N_PAD, D), jnp.float32),
            pltpu.VMEM_SHARED((N_PAD, D) if table_in_spmem else (8,),
                              jnp.float32),
            pltpu.SemaphoreType.DMA,
            pltpu.SemaphoreType.DMA,
        ],
        compiler_params=pltpu.CompilerParams(use_tc_tiling_on_sc=False),
    )
    def agg(h_hbm, src_hbm, dst_hbm, zeros_hbm, out_hbm,
            src_v, dst_v, rows_v, acc, table, gsem, ssem):
        c = lax.axis_index("c")
        s = lax.axis_index("s")
        wid = c * NS + s
        r0 = s * ROWS_PER_TILE
        pltpu.sync_copy(zeros_hbm.at[pl.ds(r0, ROWS_PER_TILE)],
                        acc.at[pl.ds(r0, ROWS_PER_TILE)])
        if table_in_spmem:
            pltpu.sync_copy(h_hbm.at[pl.ds(r0, ROWS_PER_TILE)],
                            table.at[pl.ds(r0, ROWS_PER_TILE)])
            gsrc = table
        else:
            gsrc = h_hbm
        pltpu.sync_copy(src_hbm.at[wid], src_v)
        pltpu.sync_copy(dst_hbm.at[wid], dst_v)
        plsc.subcore_barrier()

        def body(r, carry):
            gds = [pltpu.async_copy(gsrc.at[src_v.at[r * K + k]],
                                    rows_v.at[k], gsem)
                   for k in range(K)]
            sds = []
            for k in range(K):
                gds[k].wait()
                sds.append(pltpu.async_copy(rows_v.at[k],
                                            acc.at[dst_v.at[r * K + k]],
                                            ssem, add=True))
            for d in sds:
                d.wait()
            return carry

        lax.fori_loop(0, CHUNKS // K, body, 0)
        plsc.subcore_barrier()
        pltpu.sync_copy(acc.at[pl.ds(r0, ROWS_PER_TILE)],
                        out_hbm.at[c, pl.ds(r0, ROWS_PER_TILE)])

    return agg


_agg_hid = _make_agg(HID, table_in_spmem=True, K=2)
_agg_out = _make_agg(OUT, table_in_spmem=False)


_BR = 1024


def _dinv_col(deg_ref):
    deg = deg_ref[:, 0:1] + deg_ref[:, 1:2] + 1.0
    return lax.rsqrt(deg)


def _tc1_body(deg_ref, x_ref, w_ref, o_ref):
    dinv = _dinv_col(deg_ref)
    h = jnp.dot(x_ref[...], w_ref[...], preferred_element_type=jnp.float32)
    o_ref[...] = h * dinv


def _tc1(degt, xp, W1):
    return pl.pallas_call(
        _tc1_body,
        grid=(N_PAD // _BR,),
        in_specs=[
            pl.BlockSpec((_BR, 2), lambda i: (i, 0)),
            pl.BlockSpec((_BR, IN_DIM), lambda i: (i, 0)),
            pl.BlockSpec((IN_DIM, HID), lambda i: (0, 0)),
        ],
        out_specs=pl.BlockSpec((_BR, HID), lambda i: (i, 0)),
        out_shape=jax.ShapeDtypeStruct((N_PAD, HID), jnp.float32),
    )(degt, xp, W1)


def _tc2_body(p_ref, h_ref, deg_ref, b_ref, w_ref, o_ref):
    dinv = _dinv_col(deg_ref)
    agg = p_ref[0] + p_ref[1] + h_ref[...]
    z = jnp.maximum(agg * dinv + b_ref[...], 0.0)
    h2 = jnp.dot(z, w_ref[...], preferred_element_type=jnp.float32)
    o_ref[...] = h2 * dinv


def _tc2(p1, h1s, degt, b1, W2):
    return pl.pallas_call(
        _tc2_body,
        grid=(N_PAD // _BR,),
        in_specs=[
            pl.BlockSpec((NC, _BR, HID), lambda i: (0, i, 0)),
            pl.BlockSpec((_BR, HID), lambda i: (i, 0)),
            pl.BlockSpec((_BR, 2), lambda i: (i, 0)),
            pl.BlockSpec((1, HID), lambda i: (0, 0)),
            pl.BlockSpec((HID, OUT), lambda i: (0, 0)),
        ],
        out_specs=pl.BlockSpec((_BR, OUT), lambda i: (i, 0)),
        out_shape=jax.ShapeDtypeStruct((N_PAD, OUT), jnp.float32),
    )(p1, h1s, degt, b1, W2)


def _tc3_body(p_ref, h_ref, deg_ref, b_ref, o_ref):
    dinv = _dinv_col(deg_ref)
    o = (p_ref[0] + p_ref[1] + h_ref[...]) * dinv + b_ref[...]
    m = jnp.max(o, axis=1, keepdims=True)
    lse = jnp.log(jnp.sum(jnp.exp(o - m), axis=1, keepdims=True)) + m
    o_ref[...] = o - lse


def _tc3(p2, h2s, degt, b2):
    return pl.pallas_call(
        _tc3_body,
        grid=(N_PAD // _BR,),
        in_specs=[
            pl.BlockSpec((NC, _BR, OUT), lambda i: (0, i, 0)),
            pl.BlockSpec((_BR, OUT), lambda i: (i, 0)),
            pl.BlockSpec((_BR, 2), lambda i: (i, 0)),
            pl.BlockSpec((1, OUT), lambda i: (0, 0)),
        ],
        out_specs=pl.BlockSpec((_BR, OUT), lambda i: (i, 0)),
        out_shape=jax.ShapeDtypeStruct((N_PAD, OUT), jnp.float32),
    )(p2, h2s, degt, b2)


def _edge_layout(a):
    a = a.reshape(NW, E_PER_W)
    a = jnp.pad(a, ((0, 0), (0, E_PAD_W - E_PER_W)),
                constant_values=N_PAD - 1)
    return a.reshape(NW, CHUNKS, CHUNK)


@jax.jit
def kernel(x, edge_index, W1, b1, W2, b2):
    ei = edge_index.astype(jnp.int32)
    src3 = _edge_layout(ei[0])
    dst3 = _edge_layout(ei[1])
    xp = jnp.pad(x, ((0, N_PAD - N), (0, 0)))
    z1 = jnp.zeros((N_PAD,), jnp.float32)
    zh = jnp.zeros((N_PAD, HID), jnp.float32)
    zo = jnp.zeros((N_PAD, OUT), jnp.float32)

    degp = _sc_degree(dst3, z1)
    degt = degp.T
    h1s = _tc1(degt, xp, W1)
    p1 = _agg_hid(h1s, src3, dst3, zh)
    h2s = _tc2(p1, h1s, degt, b1.reshape(1, HID), W2)
    p2 = _agg_out(h2s, src3, dst3, zo)
    o = _tc3(p2, h2s, degt, b2.reshape(1, OUT))
    return o[:N]


# speedup vs baseline: 42.1005x; 1.1503x over previous
import functools

import jax
import jax.numpy as jnp
from jax import lax
from jax.experimental import pallas as pl
from jax.experimental.pallas import tpu as pltpu
from jax.experimental.pallas import tpu_sc as plsc

N = 10000
N_PAD = 10240
IN_DIM = 128
HID = 64
OUT = 16
E = 320000
NC = 2
NS = 16
NW = NC * NS
CHUNK = 128
E_PER_W = E // NW
K = 8
CHUNKS = 80
E_PAD_W = CHUNKS * CHUNK
ROWS_PER_TILE = N_PAD // NS

_mesh = plsc.VectorSubcoreMesh(core_axis_name="c", subcore_axis_name="s")


@functools.partial(
    pl.kernel,
    out_type=jax.ShapeDtypeStruct((NC, N_PAD), jnp.float32),
    mesh=_mesh,
    scratch_types=[
        pltpu.VMEM((CHUNKS, CHUNK), jnp.int32),
        pltpu.VMEM((CHUNK,), jnp.float32),
        pltpu.VMEM_SHARED((N_PAD,), jnp.float32),
        pltpu.SemaphoreType.DMA,
    ],
)
def _sc_degree(dst_hbm, zeros_hbm, out_hbm, dst_v, ones_v, acc, sem):
    c = lax.axis_index("c")
    s = lax.axis_index("s")
    wid = c * NS + s
    r0 = s * ROWS_PER_TILE
    pltpu.sync_copy(zeros_hbm.at[pl.ds(r0, ROWS_PER_TILE)],
                    acc.at[pl.ds(r0, ROWS_PER_TILE)])
    pltpu.sync_copy(dst_hbm.at[wid], dst_v)
    for i in range(CHUNK // 16):
        ones_v[pl.ds(i * 16, 16)] = jnp.full((16,), 1.0, jnp.float32)
    plsc.subcore_barrier()

    def body(r, carry):
        ds = [pltpu.async_copy(ones_v, acc.at[dst_v.at[r * K + k]], sem,
                               add=True)
              for k in range(K)]
        for d in ds:
            d.wait()
        return carry

    lax.fori_loop(0, CHUNKS // K, body, 0)
    plsc.subcore_barrier()
    pltpu.sync_copy(acc.at[pl.ds(r0, ROWS_PER_TILE)],
                    out_hbm.at[c, pl.ds(r0, ROWS_PER_TILE)])


def _make_agg(D, table_in_spmem, K=K):
    @functools.partial(
        pl.kernel,
        out_type=jax.ShapeDtypeStruct((NC, N_PAD, D), jnp.float32),
        mesh=_mesh,
        scratch_types=[
            pltpu.VMEM((CHUNKS, CHUNK), jnp.int32),
            pltpu.VMEM((CHUNKS, CHUNK), jnp.int32),
            pltpu.VMEM((K, CHUNK, D), jnp.float32),
            pltpu.VMEM_SHARED((N_PAD, D), jnp.float32),
            pltpu.VMEM_SHARED((N_PAD, D) if table_in_spmem else (8,),
                              jnp.float32),
            pltpu.SemaphoreType.DMA,
            pltpu.SemaphoreType.DMA,
        ],
        compiler_params=pltpu.CompilerParams(use_tc_tiling_on_sc=False),
    )
    def agg(h_hbm, src_hbm, dst_hbm, zeros_hbm, out_hbm,
            src_v, dst_v, rows_v, acc, table, gsem, ssem):
        c = lax.axis_index("c")
        s = lax.axis_index("s")
        wid = c * NS + s
        r0 = s * ROWS_PER_TILE
        pltpu.sync_copy(zeros_hbm.at[pl.ds(r0, ROWS_PER_TILE)],
                        acc.at[pl.ds(r0, ROWS_PER_TILE)])
        if table_in_spmem:
            pltpu.sync_copy(h_hbm.at[pl.ds(r0, ROWS_PER_TILE)],
                            table.at[pl.ds(r0, ROWS_PER_TILE)])
            gsrc = table
        else:
            gsrc = h_hbm
        pltpu.sync_copy(src_hbm.at[wid], src_v)
        pltpu.sync_copy(dst_hbm.at[wid], dst_v)
        plsc.subcore_barrier()

        def body(r, carry):
            gds = [pltpu.async_copy(gsrc.at[src_v.at[r * K + k]],
                                    rows_v.at[k], gsem)
                   for k in range(K)]
            sds = []
            for k in range(K):
                gds[k].wait()
                sds.append(pltpu.async_copy(rows_v.at[k],
                                            acc.at[dst_v.at[r * K + k]],
                                            ssem, add=True))
            for d in sds:
                d.wait()
            return carry

        lax.fori_loop(0, CHUNKS // K, body, 0)
        plsc.subcore_barrier()
        pltpu.sync_copy(acc.at[pl.ds(r0, ROWS_PER_TILE)],
                        out_hbm.at[c, pl.ds(r0, ROWS_PER_TILE)])

    return agg


_agg_hid = _make_agg(HID, table_in_spmem=True, K=2)
_agg_out = _make_agg(OUT, table_in_spmem=True)


_BR = 1024


def _dinv_col(deg_ref):
    deg = deg_ref[:, 0:1] + deg_ref[:, 1:2] + 1.0
    return lax.rsqrt(deg)


def _tc1_body(deg_ref, x_ref, w_ref, o_ref):
    dinv = _dinv_col(deg_ref)
    h = jnp.dot(x_ref[...], w_ref[...], preferred_element_type=jnp.float32)
    o_ref[...] = h * dinv


def _tc1(degt, xp, W1):
    return pl.pallas_call(
        _tc1_body,
        grid=(N_PAD // _BR,),
        in_specs=[
            pl.BlockSpec((_BR, 2), lambda i: (i, 0)),
            pl.BlockSpec((_BR, IN_DIM), lambda i: (i, 0)),
            pl.BlockSpec((IN_DIM, HID), lambda i: (0, 0)),
        ],
        out_specs=pl.BlockSpec((_BR, HID), lambda i: (i, 0)),
        out_shape=jax.ShapeDtypeStruct((N_PAD, HID), jnp.float32),
    )(degt, xp, W1)


def _tc2_body(p_ref, h_ref, deg_ref, b_ref, w_ref, o_ref):
    dinv = _dinv_col(deg_ref)
    agg = p_ref[0] + p_ref[1] + h_ref[...]
    z = jnp.maximum(agg * dinv + b_ref[...], 0.0)
    h2 = jnp.dot(z, w_ref[...], preferred_element_type=jnp.float32)
    o_ref[...] = h2 * dinv


def _tc2(p1, h1s, degt, b1, W2):
    return pl.pallas_call(
        _tc2_body,
        grid=(N_PAD // _BR,),
        in_specs=[
            pl.BlockSpec((NC, _BR, HID), lambda i: (0, i, 0)),
            pl.BlockSpec((_BR, HID), lambda i: (i, 0)),
            pl.BlockSpec((_BR, 2), lambda i: (i, 0)),
            pl.BlockSpec((1, HID), lambda i: (0, 0)),
            pl.BlockSpec((HID, OUT), lambda i: (0, 0)),
        ],
        out_specs=pl.BlockSpec((_BR, OUT), lambda i: (i, 0)),
        out_shape=jax.ShapeDtypeStruct((N_PAD, OUT), jnp.float32),
    )(p1, h1s, degt, b1, W2)


def _tc3_body(p_ref, h_ref, deg_ref, b_ref, o_ref):
    dinv = _dinv_col(deg_ref)
    o = (p_ref[0] + p_ref[1] + h_ref[...]) * dinv + b_ref[...]
    m = jnp.max(o, axis=1, keepdims=True)
    lse = jnp.log(jnp.sum(jnp.exp(o - m), axis=1, keepdims=True)) + m
    o_ref[...] = o - lse


def _tc3(p2, h2s, degt, b2):
    return pl.pallas_call(
        _tc3_body,
        grid=(N_PAD // _BR,),
        in_specs=[
            pl.BlockSpec((NC, _BR, OUT), lambda i: (0, i, 0)),
            pl.BlockSpec((_BR, OUT), lambda i: (i, 0)),
            pl.BlockSpec((_BR, 2), lambda i: (i, 0)),
            pl.BlockSpec((1, OUT), lambda i: (0, 0)),
        ],
        out_specs=pl.BlockSpec((_BR, OUT), lambda i: (i, 0)),
        out_shape=jax.ShapeDtypeStruct((N_PAD, OUT), jnp.float32),
    )(p2, h2s, degt, b2)


def _edge_layout(a):
    a = a.reshape(NW, E_PER_W)
    a = jnp.pad(a, ((0, 0), (0, E_PAD_W - E_PER_W)),
                constant_values=N_PAD - 1)
    return a.reshape(NW, CHUNKS, CHUNK)


@jax.jit
def kernel(x, edge_index, W1, b1, W2, b2):
    ei = edge_index.astype(jnp.int32)
    src3 = _edge_layout(ei[0])
    dst3 = _edge_layout(ei[1])
    xp = jnp.pad(x, ((0, N_PAD - N), (0, 0)))
    z1 = jnp.zeros((N_PAD,), jnp.float32)
    zh = jnp.zeros((N_PAD, HID), jnp.float32)
    zo = jnp.zeros((N_PAD, OUT), jnp.float32)

    degp = _sc_degree(dst3, z1)
    degt = degp.T
    h1s = _tc1(degt, xp, W1)
    p1 = _agg_hid(h1s, src3, dst3, zh)
    h2s = _tc2(p1, h1s, degt, b1.reshape(1, HID), W2)
    p2 = _agg_out(h2s, src3, dst3, zo)
    o = _tc3(p2, h2s, degt, b2.reshape(1, OUT))
    return o[:N]
